# baseline (speedup 1.0000x reference)
import jax
import jax.numpy as jnp
from jax import lax
from jax.experimental import pallas as pl
from jax.experimental.pallas import tpu as pltpu
from jax.experimental.pallas import tpu_sc as plsc

_DEPTH = 1000
_N = 26624
_LANES = 1024
_NSUB = _N * _DEPTH // _LANES
_SUB = 1000
_GRID = _NSUB // _SUB
_SCW = 25
_CHUNKS = _NSUB // 16 // _SCW
_WWORDS = _CHUNKS * 16


def _sc_index_stage(idx_hbm, ta_hbm, tb_hbm, tc_hbm, idx_v, bufa, bufb,
                    bufc, sem):
    wid = lax.axis_index("s") * 2 + lax.axis_index("c")

    @pl.when(wid < _SCW)
    def _():
        pltpu.async_copy(idx_hbm, idx_v, sem).wait()
        iota = lax.iota(jnp.int32, 16)
        base = wid * _WWORDS

        def chunk(j, _):
            sg = base + j * 16 + iota
            i = jax.lax.shift_right_logical(sg * 33555, 25)
            sl = sg - 1000 * i
            x = sl * 24
            c0 = jax.lax.shift_right_logical(x * 8389, 23)
            lstar = 1000 * (c0 + 1) - x
            qa = sg + 24 * i + c0
            qb = jnp.minimum(qa + 1, _N - 1)
            qc = jnp.minimum(qa + 2, _N - 1)
            ia = plsc.load_gather(idx_v, [qa])
            ib = plsc.load_gather(idx_v, [qb])
            ic = plsc.load_gather(idx_v, [qc])
            bufa[pl.ds(j * 16, 16)] = ia + lstar - 1000
            bufb[pl.ds(j * 16, 16)] = ib + lstar
            bufc[pl.ds(j * 16, 16)] = ic + lstar + 1000
            return 0

        lax.fori_loop(0, _CHUNKS, chunk, 0)
        pltpu.async_copy(bufa, ta_hbm.at[pl.ds(base, _WWORDS)], sem).wait()
        pltpu.async_copy(bufb, tb_hbm.at[pl.ds(base, _WWORDS)], sem).wait()
        pltpu.async_copy(bufc, tc_hbm.at[pl.ds(base, _WWORDS)], sem).wait()


def _tc_fill_block(ta_ref, tb_ref, tc_ref, val_ref, out_ref):
    ta = ta_ref[...].reshape(_SUB, 1)
    tb = tb_ref[...].reshape(_SUB, 1)
    tc = tc_ref[...].reshape(_SUB, 1)
    lane = jax.lax.broadcasted_iota(jnp.int32, (_SUB, _LANES), 1)
    m = (lane == ta) | (lane == tb) | (lane == tc)
    out_ref[...] = jnp.where(m, val_ref[1], val_ref[0])


def kernel(indices, values):
    idx_flat = indices.reshape(-1)
    tshape = jax.ShapeDtypeStruct((_NSUB,), jnp.int32)
    sc_stage = pl.kernel(
        _sc_index_stage,
        out_type=(tshape, tshape, tshape),
        mesh=plsc.VectorSubcoreMesh(core_axis_name="c", subcore_axis_name="s"),
        compiler_params=pltpu.CompilerParams(needs_layout_passes=False),
        scratch_types=[
            pltpu.VMEM((_N,), jnp.int32),
            pltpu.VMEM((_WWORDS,), jnp.int32),
            pltpu.VMEM((_WWORDS,), jnp.int32),
            pltpu.VMEM((_WWORDS,), jnp.int32),
            pltpu.SemaphoreType.DMA,
        ],
    )
    ta, tb, tc = sc_stage(idx_flat)
    t3 = (_GRID, 1, _SUB)
    tspec = pl.BlockSpec((1, 1, _SUB), lambda i: (i, 0, 0))
    out = pl.pallas_call(
        _tc_fill_block,
        grid=(_GRID,),
        in_specs=[
            tspec,
            tspec,
            tspec,
            pl.BlockSpec(memory_space=pltpu.SMEM),
        ],
        out_specs=pl.BlockSpec((_SUB, _LANES), lambda i: (i, 0)),
        out_shape=jax.ShapeDtypeStruct((_NSUB, _LANES), jnp.float32),
    )(ta.reshape(t3), tb.reshape(t3), tc.reshape(t3), values)
    return out.reshape(*indices.shape, _DEPTH)


# speedup vs baseline: 2.7318x; 2.7318x over previous
import jax
import jax.numpy as jnp
from jax.experimental import pallas as pl
from jax.experimental.pallas import tpu as pltpu

_DEPTH = 1000
_B = 32


def _one_hot_block(idx_ref, val_ref, out_ref):
    idx = idx_ref[...]
    t = idx.reshape(_B, idx.shape[1], 1)
    lane = jax.lax.broadcasted_iota(jnp.int32, (_B, idx.shape[1], _DEPTH), 2)
    out_ref[...] = jnp.where(lane == t, val_ref[1], val_ref[0])


def kernel(indices, values):
    n, m = indices.shape
    out = pl.pallas_call(
        _one_hot_block,
        grid=(n // _B,),
        in_specs=[
            pl.BlockSpec((_B, m), lambda i: (i, 0)),
            pl.BlockSpec(memory_space=pltpu.SMEM),
        ],
        out_specs=pl.BlockSpec((_B, m, _DEPTH), lambda i: (i, 0, 0)),
        out_shape=jax.ShapeDtypeStruct((n, m, _DEPTH), jnp.float32),
    )(indices, values)
    return out


# speedup vs baseline: 2.7352x; 1.0013x over previous
import jax
import jax.numpy as jnp
from jax.experimental import pallas as pl
from jax.experimental.pallas import tpu as pltpu

_DEPTH = 1000
_B = 64


def _one_hot_block(idx_ref, val_ref, out_ref):
    idx = idx_ref[...]
    t = idx.reshape(_B, idx.shape[1], 1)
    lane = jax.lax.broadcasted_iota(jnp.int32, (_B, idx.shape[1], _DEPTH), 2)
    out_ref[...] = jnp.where(lane == t, val_ref[1], val_ref[0])


def kernel(indices, values):
    n, m = indices.shape
    out = pl.pallas_call(
        _one_hot_block,
        grid=(n // _B,),
        in_specs=[
            pl.BlockSpec((_B, m), lambda i: (i, 0)),
            pl.BlockSpec(memory_space=pltpu.SMEM),
        ],
        out_specs=pl.BlockSpec((_B, m, _DEPTH), lambda i: (i, 0, 0)),
        out_shape=jax.ShapeDtypeStruct((n, m, _DEPTH), jnp.float32),
    )(indices, values)
    return out
